# Initial kernel scaffold; baseline (speedup 1.0000x reference)
import jax
import jax.numpy as jnp
from jax.experimental import pallas as pl


def kernel(p, x, W_in, W11, W12, W21, W22, W23, W31, W32, W34, W41, W42, W_agg, mlp_w1, gn_gamma, gn_beta, mlp_w2):
    raise NotImplementedError("write your pallas kernel here")


# speedup vs baseline: 8.2442x; 8.2442x over previous
import functools

import numpy as np
import jax
import jax.numpy as jnp
from jax.experimental import pallas as pl
from jax.experimental.pallas import tpu as pltpu
from jax.experimental.pallas import tpu_sc as plsc

K_PTS = 15
K_NN = 16

_INTERPRET = False

_KPTS_UNIT = np.random.RandomState(42).uniform(-1.0, 1.0, (K_PTS, 3)).astype(np.float32)


def _knn(pq, ps_t, tq):
    B, M, _ = pq.shape
    ns = ps_t.shape[2]
    grid = (B, M // tq)

    def body(pq_ref, ps_ref, idx_ref):
        b = pl.program_id(0)
        pqt = pq_ref[0]
        pst = ps_ref[0]
        q2 = None
        s2 = None
        for d in range(3):
            qd = pqt[:, d:d + 1]
            sd = pst[d:d + 1, :]
            q2 = qd * qd if q2 is None else q2 + qd * qd
            s2 = sd * sd if s2 is None else s2 + sd * sd
        e = jnp.dot(pqt.astype(jnp.bfloat16), pst.astype(jnp.bfloat16),
                    preferred_element_type=jnp.float32)
        d2 = (q2 - 2.0 * e) + s2
        lane = jax.lax.broadcasted_iota(jnp.int32, (tq, ns), 1)
        bigi = jnp.int32(2**30)
        cols = []
        cur = d2
        for _ in range(K_NN):
            m = jnp.min(cur, axis=1, keepdims=True)
            cand = jnp.where(cur == m, lane, bigi)
            am = jnp.min(cand, axis=1, keepdims=True)
            cols.append(am)
            cur = jnp.where(cand == am, jnp.float32(jnp.inf), cur)
        idxs = jnp.concatenate(cols, axis=1)
        idx_ref[0] = jnp.transpose(idxs) + b * ns

    return pl.pallas_call(
        body,
        grid=grid,
        in_specs=[
            pl.BlockSpec((1, tq, 3), lambda b, t: (b, t, 0)),
            pl.BlockSpec((1, 3, ns), lambda b, t: (b, 0, 0)),
        ],
        out_specs=pl.BlockSpec((1, K_NN, tq), lambda b, t: (b, 0, t)),
        out_shape=jax.ShapeDtypeStruct((B, K_NN, M), jnp.int32),
        interpret=_INTERPRET,
    )(pq, ps_t)


_NUM_SC = 2
_NUM_SUBCORES = 16
_NW = _NUM_SC * _NUM_SUBCORES


def _sc_gather(table, idx):
    v, d = table.shape
    bi = idx.shape[0]
    if _INTERPRET:
        return table[idx]
    b_per_w = bi // _NW
    nch = 1
    while (b_per_w // nch) * d > 96 * 1024 or (b_per_w // nch) > 8192:
        nch *= 2
    chunk = b_per_w // nch
    mesh = plsc.VectorSubcoreMesh(core_axis_name="c", subcore_axis_name="s")

    @functools.partial(
        pl.kernel,
        mesh=mesh,
        compiler_params=pltpu.CompilerParams(use_tc_tiling_on_sc=False),
        out_type=jax.ShapeDtypeStruct((bi, d), jnp.float32),
        scratch_types=[
            pltpu.VMEM((chunk,), jnp.int32),
            pltpu.VMEM((chunk, d), jnp.float32),
            pltpu.SemaphoreType.DMA,
        ],
    )
    def gk(table_hbm, idx_hbm, out_hbm, idx_v, rows_v, sem):
        wid = jax.lax.axis_index("s") * _NUM_SC + jax.lax.axis_index("c")
        base = wid * b_per_w
        for c in range(nch):
            off = base + c * chunk
            pltpu.sync_copy(idx_hbm.at[pl.ds(off, chunk)], idx_v)
            pltpu.async_copy(table_hbm.at[idx_v], rows_v, sem).wait()
            pltpu.sync_copy(rows_v, out_hbm.at[pl.ds(off, chunk)])

    return gk(table, idx)


def _kpconv(g, p_out, w_flat, radius, cin, cout, fp_out, tm):
    B, M, _ = p_out.shape
    fp_in = g.shape[2]
    grid = (B, M // tm)
    kr_t = jnp.asarray((_KPTS_UNIT * radius).T)
    inv_r = float(1.0 / radius)
    pad = fp_out - 3 - cout

    def body(g_ref, po_ref, w_ref, kr_ref, out_ref):
        po = po_ref[0]
        accs = [jnp.zeros((tm, cin), jnp.float32) for _ in range(K_PTS)]
        for j in range(K_NN):
            slab = g_ref[j]
            pn = slab[:, 0:3]
            xn = slab[:, 3:3 + cin]
            rel = pn - po
            dacc = None
            for dd in range(3):
                diff = rel[:, dd:dd + 1] - kr_ref[dd:dd + 1, :]
                sq = diff * diff
                dacc = sq if dacc is None else dacc + sq
            dist = jnp.sqrt(dacc + 1e-12)
            infl = jnp.maximum(0.0, 1.0 - dist * inv_r)
            infl = infl.astype(jnp.bfloat16).astype(jnp.float32)
            xn = xn.astype(jnp.bfloat16).astype(jnp.float32)
            for q in range(K_PTS):
                accs[q] = accs[q] + infl[:, q:q + 1] * xn
        z = jnp.concatenate(accs, axis=1)
        res = jnp.dot(z.astype(jnp.bfloat16),
                      w_ref[...].astype(jnp.bfloat16),
                      preferred_element_type=jnp.float32)
        res = jnp.where(res >= 0, res, 0.1 * res)
        parts = [po, res]
        if pad:
            parts.append(jnp.zeros((tm, pad), jnp.float32))
        out_ref[0] = jnp.concatenate(parts, axis=1)

    return pl.pallas_call(
        body,
        grid=grid,
        in_specs=[
            pl.BlockSpec((K_NN, tm, fp_in), lambda b, t: (b, t, 0)),
            pl.BlockSpec((1, tm, 3), lambda b, t: (b, t, 0)),
            pl.BlockSpec(w_flat.shape, lambda b, t: (0, 0)),
            pl.BlockSpec((3, K_PTS), lambda b, t: (0, 0)),
        ],
        out_specs=pl.BlockSpec((1, tm, fp_out), lambda b, t: (b, t, 0)),
        out_shape=jax.ShapeDtypeStruct((B, M, fp_out), jnp.float32),
        interpret=_INTERPRET,
    )(g, p_out, w_flat, kr_t)


def _aggregate_mlp(t, wa_flat, w1, gamma, beta, w2):
    B = t.shape[0]
    kr_t = jnp.asarray(_KPTS_UNIT.T)

    def body(t_ref, wa_ref, w1_ref, g_ref, b_ref, w2_ref, kr_ref, out_ref):
        encs = []
        for b in range(B):
            tb = t_ref[b]
            pb = tb[:, 0:3]
            xb = tb[:, 3:67]
            center = jnp.mean(pb, axis=0, keepdims=True)
            rel = pb - center
            dacc = None
            for dd in range(3):
                diff = rel[:, dd:dd + 1] - kr_ref[dd:dd + 1, :]
                sq = diff * diff
                dacc = sq if dacc is None else dacc + sq
            dist = jnp.sqrt(dacc + 1e-12)
            infl = jnp.maximum(0.0, 1.0 - dist)
            inflr = infl.astype(jnp.bfloat16).astype(jnp.float32)
            xbr = xb.astype(jnp.bfloat16).astype(jnp.float32)
            y = jnp.concatenate(
                [inflr[:, q:q + 1] * xbr for q in range(K_PTS)], axis=1)
            agg = jnp.sum(y, axis=0, keepdims=True)
            encs.append(jnp.dot(agg.astype(jnp.bfloat16),
                                wa_ref[...].astype(jnp.bfloat16),
                                preferred_element_type=jnp.float32))
        enc = jnp.concatenate(encs, axis=0)
        enc = jnp.where(enc >= 0, enc, 0.1 * enc)
        h = jnp.dot(enc.astype(jnp.bfloat16),
                    w1_ref[...].astype(jnp.bfloat16),
                    preferred_element_type=jnp.float32)
        outs = []
        for grp in range(4):
            sub = h[:, grp * 16:(grp + 1) * 16]
            mu = jnp.mean(sub, axis=1, keepdims=True)
            var = jnp.mean((sub - mu) * (sub - mu), axis=1, keepdims=True)
            outs.append((sub - mu) / jnp.sqrt(var + 1e-5))
        hn = jnp.concatenate(outs, axis=1) * g_ref[...] + b_ref[...]
        hn = jnp.where(hn >= 0, hn, 0.1 * hn)
        out_ref[...] = jnp.dot(hn.astype(jnp.bfloat16),
                               w2_ref[...].astype(jnp.bfloat16),
                               preferred_element_type=jnp.float32)

    return pl.pallas_call(
        body,
        out_shape=jax.ShapeDtypeStruct((B, 64), jnp.float32),
        interpret=_INTERPRET,
    )(t, wa_flat, w1, gamma, beta, w2, kr_t)


def kernel(p, x, W_in, W11, W12, W21, W22, W23, W31, W32, W34, W41, W42,
           W_agg, mlp_w1, gn_gamma, gn_beta, mlp_w2):
    B, N, _ = p.shape
    p2 = p[:, ::4]
    p3 = p2[:, ::4]
    p4 = p3[:, ::4]
    pt = jnp.swapaxes(p, 1, 2)
    p2t = jnp.swapaxes(p2, 1, 2)
    p3t = jnp.swapaxes(p3, 1, 2)
    p4t = jnp.swapaxes(p4, 1, 2)

    idx_a = _knn(p, pt, 256)
    idx_b = _knn(p2, pt, 256)
    idx_c = _knn(p2, p2t, 256)
    idx_d = _knn(p3, p2t, 128)
    idx_e = _knn(p3, p3t, 128)
    idx_f = _knn(p4, p3t, 32)
    idx_g = _knn(p4, p4t, 32)

    def wf(w):
        return w.reshape(K_PTS * w.shape[1], w.shape[2])

    t0 = jnp.concatenate(
        [p, x, p, jnp.zeros((B, N, 32 - 22), jnp.float32)], axis=-1
    ).reshape(B * N, 32)

    g0 = _sc_gather(t0, idx_a.reshape(-1)).reshape(B * K_NN, N, 32)
    t1 = _kpconv(g0, p, wf(W_in), 0.025, 19, 16, 32, 512)
    g1 = _sc_gather(t1.reshape(B * N, 32), idx_a.reshape(-1)).reshape(B * K_NN, N, 32)
    t2 = _kpconv(g1, p, wf(W11), 0.025, 16, 16, 32, 512)
    g2 = _sc_gather(t2.reshape(B * N, 32), idx_b.reshape(-1)).reshape(B * K_NN, 512, 32)
    t3 = _kpconv(g2, p2, wf(W12), 0.05, 16, 32, 48, 512)
    g3 = _sc_gather(t3.reshape(B * 512, 48), idx_c.reshape(-1)).reshape(B * K_NN, 512, 48)
    t4 = _kpconv(g3, p2, wf(W21), 0.05, 32, 32, 48, 512)
    g4 = _sc_gather(t4.reshape(B * 512, 48), idx_c.reshape(-1)).reshape(B * K_NN, 512, 48)
    t5 = _kpconv(g4, p2, wf(W22), 0.05, 32, 32, 48, 512)
    g5 = _sc_gather(t5.reshape(B * 512, 48), idx_d.reshape(-1)).reshape(B * K_NN, 128, 48)
    t6 = _kpconv(g5, p3, wf(W23), 0.1, 32, 64, 80, 128)
    g6 = _sc_gather(t6.reshape(B * 128, 80), idx_e.reshape(-1)).reshape(B * K_NN, 128, 80)
    t7 = _kpconv(g6, p3, wf(W31), 0.1, 64, 64, 80, 128)
    g7 = _sc_gather(t7.reshape(B * 128, 80), idx_e.reshape(-1)).reshape(B * K_NN, 128, 80)
    t8 = _kpconv(g7, p3, wf(W32), 0.1, 64, 64, 80, 128)
    g8 = _sc_gather(t8.reshape(B * 128, 80), idx_f.reshape(-1)).reshape(B * K_NN, 32, 80)
    t9 = _kpconv(g8, p4, wf(W34), 0.2, 64, 64, 80, 32)
    g9 = _sc_gather(t9.reshape(B * 32, 80), idx_g.reshape(-1)).reshape(B * K_NN, 32, 80)
    t10 = _kpconv(g9, p4, wf(W41), 0.2, 64, 64, 80, 32)
    g10 = _sc_gather(t10.reshape(B * 32, 80), idx_g.reshape(-1)).reshape(B * K_NN, 32, 80)
    t11 = _kpconv(g10, p4, wf(W42), 0.2, 64, 64, 80, 32)

    return _aggregate_mlp(
        t11,
        W_agg.reshape(K_PTS * 64, 64),
        mlp_w1,
        gn_gamma.reshape(1, 64),
        gn_beta.reshape(1, 64),
        mlp_w2,
    )


# speedup vs baseline: 15.6689x; 1.9006x over previous
import functools

import numpy as np
import jax
import jax.numpy as jnp
from jax.experimental import pallas as pl
from jax.experimental.pallas import tpu as pltpu
from jax.experimental.pallas import tpu_sc as plsc

K_PTS = 15
K_NN = 16

_INTERPRET = False

_KPTS_UNIT = np.random.RandomState(42).uniform(-1.0, 1.0, (K_PTS, 3)).astype(np.float32)


def _knn(pq, ps_t, tq):
    B, M, _ = pq.shape
    ns = ps_t.shape[2]
    grid = (B, M // tq)

    def body(pq_ref, ps_ref, idx_ref):
        b = pl.program_id(0)
        pqt = pq_ref[0]
        pst = ps_ref[0]
        q2 = None
        s2 = None
        for d in range(3):
            qd = pqt[:, d:d + 1]
            sd = pst[d:d + 1, :]
            q2 = qd * qd if q2 is None else q2 + qd * qd
            s2 = sd * sd if s2 is None else s2 + sd * sd
        e = jnp.dot(pqt.astype(jnp.bfloat16), pst.astype(jnp.bfloat16),
                    preferred_element_type=jnp.float32)
        d2 = (q2 - 2.0 * e) + s2
        lane = jax.lax.broadcasted_iota(jnp.int32, (tq, ns), 1)
        bigi = jnp.int32(2**30)
        cols = []
        cur = d2
        for _ in range(K_NN):
            m = jnp.min(cur, axis=1, keepdims=True)
            cand = jnp.where(cur == m, lane, bigi)
            am = jnp.min(cand, axis=1, keepdims=True)
            cols.append(am)
            cur = jnp.where(cand == am, jnp.float32(jnp.inf), cur)
        idxs = jnp.concatenate(cols, axis=1)
        idx_ref[0] = jnp.transpose(idxs) + b * ns

    return pl.pallas_call(
        body,
        grid=grid,
        in_specs=[
            pl.BlockSpec((1, tq, 3), lambda b, t: (b, t, 0)),
            pl.BlockSpec((1, 3, ns), lambda b, t: (b, 0, 0)),
        ],
        out_specs=pl.BlockSpec((1, K_NN, tq), lambda b, t: (b, 0, t)),
        out_shape=jax.ShapeDtypeStruct((B, K_NN, M), jnp.int32),
        interpret=_INTERPRET,
    )(pq, ps_t)


_NUM_SC = 2
_NUM_SUBCORES = 16
_NW = _NUM_SC * _NUM_SUBCORES


def _sc_gather(table, idx):
    v, d = table.shape
    bi = idx.shape[0]
    if _INTERPRET:
        return table[idx]
    b_per_w = bi // _NW
    nch = 1
    while (b_per_w // nch) * d > 96 * 1024 or (b_per_w // nch) > 8192:
        nch *= 2
    chunk = b_per_w // nch
    mesh = plsc.VectorSubcoreMesh(core_axis_name="c", subcore_axis_name="s")

    @functools.partial(
        pl.kernel,
        mesh=mesh,
        compiler_params=pltpu.CompilerParams(use_tc_tiling_on_sc=False),
        out_type=jax.ShapeDtypeStruct((bi, d), jnp.float32),
        scratch_types=[
            pltpu.VMEM((chunk,), jnp.int32),
            pltpu.VMEM((chunk, d), jnp.float32),
            pltpu.SemaphoreType.DMA,
        ],
    )
    def gk(table_hbm, idx_hbm, out_hbm, idx_v, rows_v, sem):
        wid = jax.lax.axis_index("s") * _NUM_SC + jax.lax.axis_index("c")
        base = wid * b_per_w
        for c in range(nch):
            off = base + c * chunk
            pltpu.sync_copy(idx_hbm.at[pl.ds(off, chunk)], idx_v)
            pltpu.async_copy(table_hbm.at[idx_v], rows_v, sem).wait()
            pltpu.sync_copy(rows_v, out_hbm.at[pl.ds(off, chunk)])

    return gk(table, idx)


def _kpconv(g, po_t, w_flat, radius, cin, cout, fp_out, tm):
    B, _, M = po_t.shape
    fp_in = g.shape[2]
    grid = (B, M // tm)
    g_t = jnp.swapaxes(g, 1, 2)
    w_t = jnp.swapaxes(w_flat, 0, 1)
    kr = jnp.asarray(_KPTS_UNIT * radius)
    inv_r = float(1.0 / radius)
    pad = fp_out - 3 - cout

    def body(g_ref, po_ref, w_ref, kr_ref, out_ref):
        po = po_ref[0]
        accs = [jnp.zeros((cin, tm), jnp.float32) for _ in range(K_PTS)]
        for j in range(K_NN):
            slab = g_ref[j]
            pn = slab[0:3, :]
            xn = slab[3:3 + cin, :]
            rel = pn - po
            dacc = None
            for dd in range(3):
                diff = rel[dd:dd + 1, :] - kr_ref[:, dd:dd + 1]
                sq = diff * diff
                dacc = sq if dacc is None else dacc + sq
            dist = jnp.sqrt(dacc + 1e-12)
            infl = jnp.maximum(0.0, 1.0 - dist * inv_r)
            infl = infl.astype(jnp.bfloat16).astype(jnp.float32)
            xn = xn.astype(jnp.bfloat16).astype(jnp.float32)
            for q in range(K_PTS):
                accs[q] = accs[q] + infl[q:q + 1, :] * xn
        z = jnp.concatenate(accs, axis=0)
        res = jnp.dot(w_ref[...].astype(jnp.bfloat16),
                      z.astype(jnp.bfloat16),
                      preferred_element_type=jnp.float32)
        res = jnp.where(res >= 0, res, 0.1 * res)
        parts = [po, res]
        if pad:
            parts.append(jnp.zeros((pad, tm), jnp.float32))
        out_ref[0] = jnp.concatenate(parts, axis=0)

    out_t = pl.pallas_call(
        body,
        grid=grid,
        in_specs=[
            pl.BlockSpec((K_NN, fp_in, tm), lambda b, t: (b, 0, t)),
            pl.BlockSpec((1, 3, tm), lambda b, t: (b, 0, t)),
            pl.BlockSpec(w_t.shape, lambda b, t: (0, 0)),
            pl.BlockSpec((K_PTS, 3), lambda b, t: (0, 0)),
        ],
        out_specs=pl.BlockSpec((1, fp_out, tm), lambda b, t: (b, 0, t)),
        out_shape=jax.ShapeDtypeStruct((B, fp_out, M), jnp.float32),
        interpret=_INTERPRET,
    )(g_t, po_t, w_t, kr)
    return out_t


def _aggregate_mlp(t, wa_flat, w1, gamma, beta, w2):
    B = t.shape[0]
    kr_t = jnp.asarray(_KPTS_UNIT.T)

    def body(t_ref, wa_ref, w1_ref, g_ref, b_ref, w2_ref, kr_ref, out_ref):
        encs = []
        for b in range(B):
            tb = t_ref[b]
            pb = tb[:, 0:3]
            xb = tb[:, 3:67]
            center = jnp.mean(pb, axis=0, keepdims=True)
            rel = pb - center
            dacc = None
            for dd in range(3):
                diff = rel[:, dd:dd + 1] - kr_ref[dd:dd + 1, :]
                sq = diff * diff
                dacc = sq if dacc is None else dacc + sq
            dist = jnp.sqrt(dacc + 1e-12)
            infl = jnp.maximum(0.0, 1.0 - dist)
            inflr = infl.astype(jnp.bfloat16).astype(jnp.float32)
            xbr = xb.astype(jnp.bfloat16).astype(jnp.float32)
            y = jnp.concatenate(
                [inflr[:, q:q + 1] * xbr for q in range(K_PTS)], axis=1)
            agg = jnp.sum(y, axis=0, keepdims=True)
            encs.append(jnp.dot(agg.astype(jnp.bfloat16),
                                wa_ref[...].astype(jnp.bfloat16),
                                preferred_element_type=jnp.float32))
        enc = jnp.concatenate(encs, axis=0)
        enc = jnp.where(enc >= 0, enc, 0.1 * enc)
        h = jnp.dot(enc.astype(jnp.bfloat16),
                    w1_ref[...].astype(jnp.bfloat16),
                    preferred_element_type=jnp.float32)
        outs = []
        for grp in range(4):
            sub = h[:, grp * 16:(grp + 1) * 16]
            mu = jnp.mean(sub, axis=1, keepdims=True)
            var = jnp.mean((sub - mu) * (sub - mu), axis=1, keepdims=True)
            outs.append((sub - mu) / jnp.sqrt(var + 1e-5))
        hn = jnp.concatenate(outs, axis=1) * g_ref[...] + b_ref[...]
        hn = jnp.where(hn >= 0, hn, 0.1 * hn)
        out_ref[...] = jnp.dot(hn.astype(jnp.bfloat16),
                               w2_ref[...].astype(jnp.bfloat16),
                               preferred_element_type=jnp.float32)

    return pl.pallas_call(
        body,
        out_shape=jax.ShapeDtypeStruct((B, 64), jnp.float32),
        interpret=_INTERPRET,
    )(t, wa_flat, w1, gamma, beta, w2, kr_t)


def kernel(p, x, W_in, W11, W12, W21, W22, W23, W31, W32, W34, W41, W42,
           W_agg, mlp_w1, gn_gamma, gn_beta, mlp_w2):
    B, N, _ = p.shape
    p2 = p[:, ::4]
    p3 = p2[:, ::4]
    p4 = p3[:, ::4]
    pt = jnp.swapaxes(p, 1, 2)
    p2t = jnp.swapaxes(p2, 1, 2)
    p3t = jnp.swapaxes(p3, 1, 2)
    p4t = jnp.swapaxes(p4, 1, 2)

    idx_a = _knn(p, pt, 256)
    idx_b = _knn(p2, pt, 256)
    idx_c = _knn(p2, p2t, 256)
    idx_d = _knn(p3, p2t, 128)
    idx_e = _knn(p3, p3t, 128)
    idx_f = _knn(p4, p3t, 32)
    idx_g = _knn(p4, p4t, 32)

    def wf(w):
        return w.reshape(K_PTS * w.shape[1], w.shape[2])

    t0 = jnp.concatenate(
        [p, x, p, jnp.zeros((B, N, 32 - 22), jnp.float32)], axis=-1
    ).reshape(B * N, 32)

    def tab(t_t):
        fp = t_t.shape[1]
        return jnp.swapaxes(t_t, 1, 2).reshape(-1, fp)

    g0 = _sc_gather(t0, idx_a.reshape(-1)).reshape(B * K_NN, N, 32)
    t1 = _kpconv(g0, pt, wf(W_in), 0.025, 19, 16, 32, 512)
    g1 = _sc_gather(tab(t1), idx_a.reshape(-1)).reshape(B * K_NN, N, 32)
    t2 = _kpconv(g1, pt, wf(W11), 0.025, 16, 16, 32, 512)
    g2 = _sc_gather(tab(t2), idx_b.reshape(-1)).reshape(B * K_NN, 512, 32)
    t3 = _kpconv(g2, p2t, wf(W12), 0.05, 16, 32, 48, 512)
    g3 = _sc_gather(tab(t3), idx_c.reshape(-1)).reshape(B * K_NN, 512, 48)
    t4 = _kpconv(g3, p2t, wf(W21), 0.05, 32, 32, 48, 512)
    g4 = _sc_gather(tab(t4), idx_c.reshape(-1)).reshape(B * K_NN, 512, 48)
    t5 = _kpconv(g4, p2t, wf(W22), 0.05, 32, 32, 48, 512)
    g5 = _sc_gather(tab(t5), idx_d.reshape(-1)).reshape(B * K_NN, 128, 48)
    t6 = _kpconv(g5, p3t, wf(W23), 0.1, 32, 64, 80, 128)
    g6 = _sc_gather(tab(t6), idx_e.reshape(-1)).reshape(B * K_NN, 128, 80)
    t7 = _kpconv(g6, p3t, wf(W31), 0.1, 64, 64, 80, 128)
    g7 = _sc_gather(tab(t7), idx_e.reshape(-1)).reshape(B * K_NN, 128, 80)
    t8 = _kpconv(g7, p3t, wf(W32), 0.1, 64, 64, 80, 128)
    g8 = _sc_gather(tab(t8), idx_f.reshape(-1)).reshape(B * K_NN, 32, 80)
    t9 = _kpconv(g8, p4t, wf(W34), 0.2, 64, 64, 80, 32)
    g9 = _sc_gather(tab(t9), idx_g.reshape(-1)).reshape(B * K_NN, 32, 80)
    t10 = _kpconv(g9, p4t, wf(W41), 0.2, 64, 64, 80, 32)
    g10 = _sc_gather(tab(t10), idx_g.reshape(-1)).reshape(B * K_NN, 32, 80)
    t11 = _kpconv(g10, p4t, wf(W42), 0.2, 64, 64, 80, 32)

    return _aggregate_mlp(
        jnp.swapaxes(t11, 1, 2),
        W_agg.reshape(K_PTS * 64, 64),
        mlp_w1,
        gn_gamma.reshape(1, 64),
        gn_beta.reshape(1, 64),
        mlp_w2,
    )


# speedup vs baseline: 17.3561x; 1.1077x over previous
import functools

import numpy as np
import jax
import jax.numpy as jnp
from jax.experimental import pallas as pl
from jax.experimental.pallas import tpu as pltpu
from jax.experimental.pallas import tpu_sc as plsc

K_PTS = 15
K_NN = 16

_KPTS_UNIT = np.random.RandomState(42).uniform(-1.0, 1.0, (K_PTS, 3)).astype(np.float32)


def _knn(pq, ps_t, tq):
    B, M, _ = pq.shape
    ns = ps_t.shape[2]
    grid = (B, M // tq)

    def body(pq_ref, ps_ref, idx_ref):
        b = pl.program_id(0)
        pqt = pq_ref[0]
        pst = ps_ref[0]
        q2 = None
        s2 = None
        for d in range(3):
            qd = pqt[:, d:d + 1]
            sd = pst[d:d + 1, :]
            q2 = qd * qd if q2 is None else q2 + qd * qd
            s2 = sd * sd if s2 is None else s2 + sd * sd
        e = jnp.dot(pqt.astype(jnp.bfloat16), pst.astype(jnp.bfloat16),
                    preferred_element_type=jnp.float32)
        d2 = (q2 - 2.0 * e) + s2
        lane = jax.lax.broadcasted_iota(jnp.int32, (tq, ns), 1)
        bigi = jnp.int32(2**30)
        cols = []
        cur = d2
        for _ in range(K_NN):
            m = jnp.min(cur, axis=1, keepdims=True)
            cand = jnp.where(cur == m, lane, bigi)
            am = jnp.min(cand, axis=1, keepdims=True)
            cols.append(am)
            cur = jnp.where(cand == am, jnp.float32(jnp.inf), cur)
        idxs = jnp.concatenate(cols, axis=1)
        idx_ref[0] = jnp.transpose(idxs) + b * ns

    return pl.pallas_call(
        body,
        grid=grid,
        in_specs=[
            pl.BlockSpec((1, tq, 3), lambda b, t: (b, t, 0)),
            pl.BlockSpec((1, 3, ns), lambda b, t: (b, 0, 0)),
        ],
        out_specs=pl.BlockSpec((1, K_NN, tq), lambda b, t: (b, 0, t)),
        out_shape=jax.ShapeDtypeStruct((B, K_NN, M), jnp.int32),
    )(pq, ps_t)


_NUM_SC = 2
_NUM_SUBCORES = 16
_NW = _NUM_SC * _NUM_SUBCORES


def _sc_gather(table, idx):
    v, d = table.shape
    bi = idx.shape[0]
    b_per_w = bi // _NW
    nch = 1
    while (b_per_w // nch) * d > 96 * 1024 or (b_per_w // nch) > 8192:
        nch *= 2
    chunk = b_per_w // nch
    mesh = plsc.VectorSubcoreMesh(core_axis_name="c", subcore_axis_name="s")

    @functools.partial(
        pl.kernel,
        mesh=mesh,
        compiler_params=pltpu.CompilerParams(use_tc_tiling_on_sc=False),
        out_type=jax.ShapeDtypeStruct((bi, d), jnp.float32),
        scratch_types=[
            pltpu.VMEM((chunk,), jnp.int32),
            pltpu.VMEM((chunk, d), jnp.float32),
            pltpu.SemaphoreType.DMA,
        ],
    )
    def gk(table_hbm, idx_hbm, out_hbm, idx_v, rows_v, sem):
        wid = jax.lax.axis_index("s") * _NUM_SC + jax.lax.axis_index("c")
        base = wid * b_per_w
        for c in range(nch):
            off = base + c * chunk
            pltpu.sync_copy(idx_hbm.at[pl.ds(off, chunk)], idx_v)
            pltpu.async_copy(table_hbm.at[idx_v], rows_v, sem).wait()
            pltpu.sync_copy(rows_v, out_hbm.at[pl.ds(off, chunk)])

    return gk(table, idx)


def _kpconv(g, po_t, w_flat, radius, cin, cout, fp_out, tm):
    B, _, M = po_t.shape
    fp_in = g.shape[2]
    grid = (B, M // tm)
    g_t = jnp.swapaxes(g, 1, 2)
    w_t = jnp.swapaxes(w_flat, 0, 1)
    kr = jnp.asarray(_KPTS_UNIT * radius)
    inv_r = float(1.0 / radius)
    pad = fp_out - 3 - cout

    def body(g_ref, po_ref, w_ref, kr_ref, out_ref):
        po = po_ref[0]
        accs = [jnp.zeros((cin, tm), jnp.float32) for _ in range(K_PTS)]
        for j in range(K_NN):
            slab = g_ref[j]
            pn = slab[0:3, :]
            xn = slab[3:3 + cin, :]
            rel = pn - po
            dacc = None
            for dd in range(3):
                diff = rel[dd:dd + 1, :] - kr_ref[:, dd:dd + 1]
                sq = diff * diff
                dacc = sq if dacc is None else dacc + sq
            dist = jnp.sqrt(dacc + 1e-12)
            infl = jnp.maximum(0.0, 1.0 - dist * inv_r)
            infl = infl.astype(jnp.bfloat16).astype(jnp.float32)
            xn = xn.astype(jnp.bfloat16).astype(jnp.float32)
            for q in range(K_PTS):
                accs[q] = accs[q] + infl[q:q + 1, :] * xn
        z = jnp.concatenate(accs, axis=0)
        res = jnp.dot(w_ref[...].astype(jnp.bfloat16),
                      z.astype(jnp.bfloat16),
                      preferred_element_type=jnp.float32)
        res = jnp.where(res >= 0, res, 0.1 * res)
        parts = [po, res]
        if pad:
            parts.append(jnp.zeros((pad, tm), jnp.float32))
        out_ref[0] = jnp.concatenate(parts, axis=0)

    out_t = pl.pallas_call(
        body,
        grid=grid,
        in_specs=[
            pl.BlockSpec((K_NN, fp_in, tm), lambda b, t: (b, 0, t)),
            pl.BlockSpec((1, 3, tm), lambda b, t: (b, 0, t)),
            pl.BlockSpec(w_t.shape, lambda b, t: (0, 0)),
            pl.BlockSpec((K_PTS, 3), lambda b, t: (0, 0)),
        ],
        out_specs=pl.BlockSpec((1, fp_out, tm), lambda b, t: (b, 0, t)),
        out_shape=jax.ShapeDtypeStruct((B, fp_out, M), jnp.float32),
    )(g_t, po_t, w_t, kr)
    return out_t


def _aggregate_mlp(t, wa_flat, w1, gamma, beta, w2):
    B = t.shape[0]
    kr_t = jnp.asarray(_KPTS_UNIT.T)

    def body(t_ref, wa_ref, w1_ref, g_ref, b_ref, w2_ref, kr_ref, out_ref):
        encs = []
        for b in range(B):
            tb = t_ref[b]
            pb = tb[:, 0:3]
            xb = tb[:, 3:67]
            center = jnp.mean(pb, axis=0, keepdims=True)
            rel = pb - center
            dacc = None
            for dd in range(3):
                diff = rel[:, dd:dd + 1] - kr_ref[dd:dd + 1, :]
                sq = diff * diff
                dacc = sq if dacc is None else dacc + sq
            dist = jnp.sqrt(dacc + 1e-12)
            infl = jnp.maximum(0.0, 1.0 - dist)
            inflr = infl.astype(jnp.bfloat16).astype(jnp.float32)
            xbr = xb.astype(jnp.bfloat16).astype(jnp.float32)
            y = jnp.concatenate(
                [inflr[:, q:q + 1] * xbr for q in range(K_PTS)], axis=1)
            agg = jnp.sum(y, axis=0, keepdims=True)
            encs.append(jnp.dot(agg.astype(jnp.bfloat16),
                                wa_ref[...].astype(jnp.bfloat16),
                                preferred_element_type=jnp.float32))
        enc = jnp.concatenate(encs, axis=0)
        enc = jnp.where(enc >= 0, enc, 0.1 * enc)
        h = jnp.dot(enc.astype(jnp.bfloat16),
                    w1_ref[...].astype(jnp.bfloat16),
                    preferred_element_type=jnp.float32)
        outs = []
        for grp in range(4):
            sub = h[:, grp * 16:(grp + 1) * 16]
            mu = jnp.mean(sub, axis=1, keepdims=True)
            var = jnp.mean((sub - mu) * (sub - mu), axis=1, keepdims=True)
            outs.append((sub - mu) / jnp.sqrt(var + 1e-5))
        hn = jnp.concatenate(outs, axis=1) * g_ref[...] + b_ref[...]
        hn = jnp.where(hn >= 0, hn, 0.1 * hn)
        out_ref[...] = jnp.dot(hn.astype(jnp.bfloat16),
                               w2_ref[...].astype(jnp.bfloat16),
                               preferred_element_type=jnp.float32)

    return pl.pallas_call(
        body,
        out_shape=jax.ShapeDtypeStruct((B, 64), jnp.float32),
    )(t, wa_flat, w1, gamma, beta, w2, kr_t)


def kernel(p, x, W_in, W11, W12, W21, W22, W23, W31, W32, W34, W41, W42,
           W_agg, mlp_w1, gn_gamma, gn_beta, mlp_w2):
    B, N, _ = p.shape
    p2 = p[:, ::4]
    p3 = p2[:, ::4]
    p4 = p3[:, ::4]
    pt = jnp.swapaxes(p, 1, 2)
    p2t = jnp.swapaxes(p2, 1, 2)
    p3t = jnp.swapaxes(p3, 1, 2)
    p4t = jnp.swapaxes(p4, 1, 2)

    idx_a = _knn(p, pt, 256)
    idx_c = _knn(p2, p2t, 256)
    idx_e = _knn(p3, p3t, 128)
    idx_g = _knn(p4, p4t, 32)
    idx_b = idx_a[:, :, ::4]
    idx_d = idx_c[:, :, ::4]
    idx_f = idx_e[:, :, ::4]

    def wf(w):
        return w.reshape(K_PTS * w.shape[1], w.shape[2])

    t0 = jnp.concatenate(
        [p, x, p, jnp.zeros((B, N, 32 - 22), jnp.float32)], axis=-1
    ).reshape(B * N, 32)

    def tab(t_t):
        fp = t_t.shape[1]
        return jnp.swapaxes(t_t, 1, 2).reshape(-1, fp)

    g0 = _sc_gather(t0, idx_a.reshape(-1)).reshape(B * K_NN, N, 32)
    t1 = _kpconv(g0, pt, wf(W_in), 0.025, 19, 16, 32, 512)
    g1 = _sc_gather(tab(t1), idx_a.reshape(-1)).reshape(B * K_NN, N, 32)
    t2 = _kpconv(g1, pt, wf(W11), 0.025, 16, 16, 32, 512)
    g2 = _sc_gather(tab(t2), idx_b.reshape(-1)).reshape(B * K_NN, 512, 32)
    t3 = _kpconv(g2, p2t, wf(W12), 0.05, 16, 32, 48, 512)
    g3 = _sc_gather(tab(t3), idx_c.reshape(-1)).reshape(B * K_NN, 512, 48)
    t4 = _kpconv(g3, p2t, wf(W21), 0.05, 32, 32, 48, 512)
    g4 = _sc_gather(tab(t4), idx_c.reshape(-1)).reshape(B * K_NN, 512, 48)
    t5 = _kpconv(g4, p2t, wf(W22), 0.05, 32, 32, 48, 512)
    g5 = _sc_gather(tab(t5), idx_d.reshape(-1)).reshape(B * K_NN, 128, 48)
    t6 = _kpconv(g5, p3t, wf(W23), 0.1, 32, 64, 80, 128)
    g6 = _sc_gather(tab(t6), idx_e.reshape(-1)).reshape(B * K_NN, 128, 80)
    t7 = _kpconv(g6, p3t, wf(W31), 0.1, 64, 64, 80, 128)
    g7 = _sc_gather(tab(t7), idx_e.reshape(-1)).reshape(B * K_NN, 128, 80)
    t8 = _kpconv(g7, p3t, wf(W32), 0.1, 64, 64, 80, 128)
    g8 = _sc_gather(tab(t8), idx_f.reshape(-1)).reshape(B * K_NN, 32, 80)
    t9 = _kpconv(g8, p4t, wf(W34), 0.2, 64, 64, 80, 32)
    g9 = _sc_gather(tab(t9), idx_g.reshape(-1)).reshape(B * K_NN, 32, 80)
    t10 = _kpconv(g9, p4t, wf(W41), 0.2, 64, 64, 80, 32)
    g10 = _sc_gather(tab(t10), idx_g.reshape(-1)).reshape(B * K_NN, 32, 80)
    t11 = _kpconv(g10, p4t, wf(W42), 0.2, 64, 64, 80, 32)

    return _aggregate_mlp(
        jnp.swapaxes(t11, 1, 2),
        W_agg.reshape(K_PTS * 64, 64),
        mlp_w1,
        gn_gamma.reshape(1, 64),
        gn_beta.reshape(1, 64),
        mlp_w2,
    )
